# baseline (speedup 1.0000x reference)
import jax
import jax.numpy as jnp
from jax import lax
from jax.experimental import pallas as pl
from jax.experimental.pallas import tpu as pltpu
from jax.experimental.pallas import tpu_sc as plsc

_B = 25
_N = 25
_SS = 25
_S = 5
_CW = 80 // _S
_NP = _B * _N
_HALF = 20
_T0 = 1920
_BUF = 2560


def _sq(x):
    return x * x


def _body(buf_hbm, out_hbm, buf_v, col_v, out_v, sem):
    wid = lax.axis_index("c") * 16 + lax.axis_index("s")

    @pl.when(wid == 0)
    def _():
        cp = pltpu.async_copy(buf_hbm, buf_v, sem)
        lane = lax.broadcasted_iota(jnp.int32, (16,), 0)
        zero16 = jnp.zeros((16,), jnp.float32)
        col_v[pl.ds(0, 16)] = zero16
        col_v[pl.ds(16, 16)] = zero16
        col_v[pl.ds(32, 16)] = zero16
        col_v[pl.ds(48, 16)] = zero16
        cp.wait()

        def coords_at(idx):
            tp = plsc.bitcast(plsc.load_gather(buf_v, [_T0 + idx]), jnp.int32)
            return (tp >> 8) + 14, (tp & 255) + 14

        def items(p):
            valid = p < _NP
            pp = jnp.minimum(p, _NP - 1)
            b = pp // _N
            a, bb = coords_at(pp)
            tx = (a % _CW).astype(jnp.float32) * (_S / 80.0)
            ty = (bb % _CW).astype(jnp.float32) * (_S / 80.0)
            cell = (a // _CW) * _S + (bb // _CW)
            cbase = b * 75 + 3 * cell
            px = plsc.load_gather(buf_v, [cbase + 1])
            py = plsc.load_gather(buf_v, [cbase + 2])
            conf = plsc.load_gather(buf_v, [b * 75 + 3 * (pp % _N)])
            dval = _sq(px - tx) + _sq(py - ty)
            dval = jnp.where(valid, dval, 0.0)
            csq = jnp.where(valid, 0.5 * conf * conf, 0.0)
            return dval, csq

        acc_a = zero16
        acc_b = zero16
        for i in range(_HALF):
            p_a = i * 16 + lane
            p_b = (_HALF + i) * 16 + lane
            dval_a, csq_a = items(p_a)
            dval_b, csq_b = items(p_b)
            plsc.addupdate_scatter(col_v, [p_a % _N], dval_a)
            plsc.addupdate_scatter(col_v, [p_b % _N + 32], dval_b)
            acc_a = acc_a + csq_a
            acc_b = acc_b + csq_b
        conf_sum = jnp.sum(acc_a + acc_b)

        r0 = lane
        r1 = jnp.minimum(lane + 16, _B - 1)
        valid1 = (lane + 16) < _B

        def cell_at(rv, j):
            a, bb = coords_at(rv * _N + j)
            return (a // _CW) * _S + (bb // _CW)

        mn0 = mx0 = cell_at(r0, 0)
        mn1 = mx1 = cell_at(r1, 0)
        for j in range(1, _N):
            ca = cell_at(r0, j)
            cb = cell_at(r1, j)
            mn0 = jnp.minimum(mn0, ca)
            mx0 = jnp.maximum(mx0, ca)
            mn1 = jnp.minimum(mn1, cb)
            mx1 = jnp.maximum(mx1, cb)

        def corr(rv, mn, mx, vmask):
            base = rv * 75 + 3 * mn
            conf0 = plsc.load_gather(buf_v, [base])
            px0 = plsc.load_gather(buf_v, [base + 1])
            py0 = plsc.load_gather(buf_v, [base + 2])
            a24, b24 = coords_at(rv * _N + (_N - 1))
            txs = (a24 % _CW).astype(jnp.float32)
            tys = (b24 % _CW).astype(jnp.float32)
            dx = jnp.abs(px0 * 16.0 - txs)
            dy = jnp.abs(py0 * 16.0 - tys)
            x1 = jnp.maximum(28.0 - 2.0 * dx, 0.0)
            y1 = jnp.maximum(28.0 - 2.0 * dy, 0.0)
            iou = (x1 * y1) / ((28.0 + dx) * (28.0 + dy))
            cval = _sq(conf0 - iou) - 0.5 * conf0 * conf0
            cval = jnp.where(mn == mx, cval, 0.0)
            return jnp.where(vmask, cval, 0.0)

        csum = jnp.sum(corr(r0, mn0, mx0, lane < _B)
                       + corr(r1, mn1, mx1, valid1))

        p_mean = (conf_sum + csum) * (1.0 / float(_B * _SS))
        out_v[pl.ds(0, 16)] = (col_v[pl.ds(0, 16)] + col_v[pl.ds(32, 16)]) \
            * (5.0 / _B) + p_mean
        out_v[pl.ds(16, 16)] = (col_v[pl.ds(16, 16)] + col_v[pl.ds(48, 16)]) \
            * (5.0 / _B) + p_mean
        pltpu.sync_copy(out_v, out_hbm)


def kernel(pred, truth, S=5):
    pred_flat = jnp.pad(pred.reshape(-1), (0, _T0 - _B * 75))
    packed = (truth[:, :, 0] * 256 + truth[:, :, 1]).reshape(-1)
    packed = jnp.pad(packed, (0, _BUF - _T0 - _NP)).astype(jnp.int32)
    buf = jnp.concatenate(
        [pred_flat, lax.bitcast_convert_type(packed, jnp.float32)])
    mesh = plsc.VectorSubcoreMesh(core_axis_name="c", subcore_axis_name="s",
                                  num_cores=1)
    out = pl.kernel(
        _body,
        mesh=mesh,
        compiler_params=pltpu.CompilerParams(needs_layout_passes=False),
        out_type=jax.ShapeDtypeStruct((32,), jnp.float32),
        scratch_types=[
            pltpu.VMEM((_BUF,), jnp.float32),
            pltpu.VMEM((64,), jnp.float32),
            pltpu.VMEM((32,), jnp.float32),
            pltpu.SemaphoreType.DMA,
        ],
    )(buf)
    return out[:_SS]


# speedup vs baseline: 1.1142x; 1.1142x over previous
import jax
import jax.numpy as jnp
from jax import lax
from jax.experimental import pallas as pl
from jax.experimental.pallas import tpu as pltpu
from jax.experimental.pallas import tpu_sc as plsc

_B = 25
_N = 25
_SS = 25
_S = 5
_CW = 80 // _S
_NP = _B * _N
_HALF = 20
_T0 = 1920
_BUF = 2560


def _sq(x):
    return x * x


def _body(buf_hbm, out_hbm, buf_v, col_v, out_v, sem):
    wid = lax.axis_index("c") * 16 + lax.axis_index("s")

    @pl.when(wid == 0)
    def _():
        cp = pltpu.async_copy(buf_hbm, buf_v, sem)
        lane = lax.broadcasted_iota(jnp.int32, (16,), 0)
        zero16 = jnp.zeros((16,), jnp.float32)
        col_v[pl.ds(0, 16)] = zero16
        col_v[pl.ds(16, 16)] = zero16
        col_v[pl.ds(32, 16)] = zero16
        col_v[pl.ds(48, 16)] = zero16
        cp.wait()

        def coords_at(idx):
            tp = plsc.load_gather(buf_v, [_T0 + idx])
            return (tp >> 8) + 14, (tp & 255) + 14

        def fgather(idx):
            return plsc.bitcast(plsc.load_gather(buf_v, [idx]), jnp.float32)

        def items(p):
            valid = p < _NP
            pp = jnp.minimum(p, _NP - 1)
            b = pp // _N
            a, bb = coords_at(pp)
            tx = (a % _CW).astype(jnp.float32) * (_S / 80.0)
            ty = (bb % _CW).astype(jnp.float32) * (_S / 80.0)
            cell = (a // _CW) * _S + (bb // _CW)
            cbase = b * 75 + 3 * cell
            px = fgather(cbase + 1)
            py = fgather(cbase + 2)
            conf = fgather(b * 75 + 3 * (pp % _N))
            dval = _sq(px - tx) + _sq(py - ty)
            dval = jnp.where(valid, dval, 0.0)
            csq = jnp.where(valid, 0.5 * conf * conf, 0.0)
            return dval, csq

        def chunk(i, accs):
            acc_a, acc_b = accs
            p_a = i * 16 + lane
            p_b = (_HALF + i) * 16 + lane
            dval_a, csq_a = items(p_a)
            dval_b, csq_b = items(p_b)
            plsc.addupdate_scatter(col_v, [p_a % _N], dval_a)
            plsc.addupdate_scatter(col_v, [p_b % _N + 32], dval_b)
            return acc_a + csq_a, acc_b + csq_b

        acc_a, acc_b = lax.fori_loop(0, _HALF, chunk, (zero16, zero16))
        conf_sum = jnp.sum(acc_a + acc_b)

        r0 = lane
        r1 = jnp.minimum(lane + 16, _B - 1)
        valid1 = (lane + 16) < _B

        def cell_at(rv, j):
            a, bb = coords_at(rv * _N + j)
            return (a // _CW) * _S + (bb // _CW)

        c00 = cell_at(r0, 0)
        c10 = cell_at(r1, 0)

        def jstep(j, carry):
            mn0, mx0, mn1, mx1 = carry
            ca = cell_at(r0, j)
            cb = cell_at(r1, j)
            return (jnp.minimum(mn0, ca), jnp.maximum(mx0, ca),
                    jnp.minimum(mn1, cb), jnp.maximum(mx1, cb))

        mn0, mx0, mn1, mx1 = lax.fori_loop(1, _N, jstep, (c00, c00, c10, c10))

        def corr(rv, mn, mx, vmask):
            base = rv * 75 + 3 * mn
            conf0 = fgather(base)
            px0 = fgather(base + 1)
            py0 = fgather(base + 2)
            a24, b24 = coords_at(rv * _N + (_N - 1))
            txs = (a24 % _CW).astype(jnp.float32)
            tys = (b24 % _CW).astype(jnp.float32)
            dx = jnp.abs(px0 * 16.0 - txs)
            dy = jnp.abs(py0 * 16.0 - tys)
            x1 = jnp.maximum(28.0 - 2.0 * dx, 0.0)
            y1 = jnp.maximum(28.0 - 2.0 * dy, 0.0)
            iou = (x1 * y1) / ((28.0 + dx) * (28.0 + dy))
            cval = _sq(conf0 - iou) - 0.5 * conf0 * conf0
            cval = jnp.where(mn == mx, cval, 0.0)
            return jnp.where(vmask, cval, 0.0)

        csum = jnp.sum(corr(r0, mn0, mx0, lane < _B)
                       + corr(r1, mn1, mx1, valid1))

        p_mean = (conf_sum + csum) * (1.0 / float(_B * _SS))
        out_v[pl.ds(0, 16)] = (col_v[pl.ds(0, 16)] + col_v[pl.ds(32, 16)]) \
            * (5.0 / _B) + p_mean
        out_v[pl.ds(16, 16)] = (col_v[pl.ds(16, 16)] + col_v[pl.ds(48, 16)]) \
            * (5.0 / _B) + p_mean
        pltpu.sync_copy(out_v, out_hbm)


def kernel(pred, truth, S=5):
    pred_flat = jnp.pad(pred.reshape(-1), (0, _T0 - _B * 75))
    packed = (truth[:, :, 0] * 256 + truth[:, :, 1]).reshape(-1)
    packed = jnp.pad(packed, (0, _BUF - _T0 - _NP)).astype(jnp.int32)
    buf = jnp.concatenate(
        [lax.bitcast_convert_type(pred_flat, jnp.int32), packed])
    mesh = plsc.VectorSubcoreMesh(core_axis_name="c", subcore_axis_name="s",
                                  num_cores=1)
    out = pl.kernel(
        _body,
        mesh=mesh,
        compiler_params=pltpu.CompilerParams(needs_layout_passes=False),
        out_type=jax.ShapeDtypeStruct((32,), jnp.float32),
        scratch_types=[
            pltpu.VMEM((_BUF,), jnp.int32),
            pltpu.VMEM((64,), jnp.float32),
            pltpu.VMEM((32,), jnp.float32),
            pltpu.SemaphoreType.DMA,
        ],
    )(buf)
    return out[:_SS]
